# baseline (speedup 1.0000x reference)
import functools

import jax
import jax.numpy as jnp
from jax import lax
from jax.experimental import pallas as pl
from jax.experimental.pallas import tpu as pltpu
from jax.experimental.pallas import tpu_sc as plsc

_T = 8192
_D = 256
_H = 512
_BLK = 512

_NC, _NS = 2, 16
_NW = _NC * _NS
_PERW = _T // _NW
_CH = 128
_NCH = _PERW // _CH


def _pack_bf16_pairs(x):
    xb = x.astype(jnp.bfloat16)
    h = x.shape[1] // 2
    lo = jax.lax.bitcast_convert_type(xb[:, :h], jnp.uint16).astype(jnp.int32)
    hi = jax.lax.bitcast_convert_type(xb[:, h:], jnp.uint16).astype(jnp.int32)
    return lo | (hi << 16)


def _unpack_pairs(x):
    lo = jax.lax.bitcast_convert_type(x << 16, jnp.float32)
    hi = jax.lax.bitcast_convert_type(x & jnp.int32(-65536), jnp.float32)
    return jnp.concatenate([lo, hi], axis=1)


_NB = 4


def _sc_gather_scatter4(op_emb, machine_emb, agv_emb, idxmat, pos):
    mesh = plsc.VectorSubcoreMesh(core_axis_name="c", subcore_axis_name="s")

    @functools.partial(
        pl.kernel, mesh=mesh,
        out_type=(jax.ShapeDtypeStruct((_T, _D // 2), jnp.int32),) * 4,
        scratch_types=[
            pltpu.VMEM((4, _PERW), jnp.int32),
            pltpu.VMEM((_CH,), jnp.int32),
            pltpu.VMEM((_CH,), jnp.int32),
        ] + [pltpu.VMEM((_CH, _D // 2), jnp.int32)] * _NB
          + [pltpu.SemaphoreType.DMA] * (2 * _NB),
    )
    def k(op_t, mach_t, agv_t, idxm_h, pos_h,
          oa, om, of_, ot,
          idxm_v, pos0, pos1, *bufs_and_sems):
        rowsb = bufs_and_sems[:_NB]
        sg = bufs_and_sems[_NB:2 * _NB]
        ss = bufs_and_sems[2 * _NB:3 * _NB]
        wid = lax.axis_index("s") * _NC + lax.axis_index("c")
        base = wid * _PERW
        posb = (pos0, pos1)
        pltpu.sync_copy(idxm_h.at[:, pl.ds(base, _PERW)], idxm_v)
        for c in range(_NCH):
            pltpu.sync_copy(pos_h.at[pl.ds(base + c * _CH, _CH)], posb[c])
        tabs = (agv_t, mach_t, op_t, op_t)
        outs = (oa, om, of_, ot)
        steps = [(c, f) for c in range(_NCH) for f in range(4)]
        n = len(steps)
        pend_g = [None] * _NB
        pend_s = [None] * _NB

        def start_gather(s):
            c, f = steps[s]
            b = s % _NB
            pend_g[b] = pltpu.async_copy(
                tabs[f].at[idxm_v.at[f, pl.ds(c * _CH, _CH)]], rowsb[b], sg[b])

        for s in range(min(3, n)):
            start_gather(s)
        for s in range(n):
            c, f = steps[s]
            b = s % _NB
            pend_g[b].wait()
            pend_s[b] = pltpu.async_copy(rowsb[b], outs[f].at[posb[c]], ss[b])
            ns = s + 3
            if ns < n:
                nb = ns % _NB
                if pend_s[nb] is not None:
                    pend_s[nb].wait()
                    pend_s[nb] = None
                start_gather(ns)
        for b in range(_NB):
            if pend_s[b] is not None:
                pend_s[b].wait()

    return k(op_emb, machine_emb, agv_emb, idxmat, pos)


def _sc_gather_rows(table, idx):
    mesh = plsc.VectorSubcoreMesh(core_axis_name="c", subcore_axis_name="s")

    @functools.partial(
        pl.kernel, mesh=mesh,
        out_type=jax.ShapeDtypeStruct((_T, _D), jnp.float32),
        scratch_types=[
            pltpu.VMEM((_PERW,), jnp.int32),
            pltpu.VMEM((_CH, _D), jnp.float32),
            pltpu.VMEM((_CH, _D), jnp.float32),
            pltpu.SemaphoreType.DMA,
            pltpu.SemaphoreType.DMA,
            pltpu.SemaphoreType.DMA,
            pltpu.SemaphoreType.DMA,
        ],
    )
    def k(tab, ih, oh, idx_v, r0, r1, sg0, sg1, sw0, sw1):
        wid = lax.axis_index("s") * _NC + lax.axis_index("c")
        base = wid * _PERW
        pltpu.sync_copy(ih.at[pl.ds(base, _PERW)], idx_v)
        g0 = pltpu.async_copy(tab.at[idx_v.at[pl.ds(0, _CH)]], r0, sg0)
        g1 = pltpu.async_copy(tab.at[idx_v.at[pl.ds(_CH, _CH)]], r1, sg1)
        g0.wait()
        w0 = pltpu.async_copy(r0, oh.at[pl.ds(base, _CH), :], sw0)
        g1.wait()
        w1 = pltpu.async_copy(r1, oh.at[pl.ds(base + _CH, _CH), :], sw1)
        w0.wait()
        w1.wait()

    return k(table, idx)


def _mlp_body(bnd_ref, a_ref, m_ref, f_ref, t_ref,
              pwa, pwf, pwt, pwm, pb1, pw2, pb2,
              twa, twm, tb1, tw2, tb2,
              mwa, mwm, mb1, mw2, mb2, wait_ref, o_ref):
    start = pl.program_id(0) * _BLK
    end = start + _BLK
    b1, b2, b3 = bnd_ref[0], bnd_ref[1], bnd_ref[2]
    rows = start + lax.broadcasted_iota(jnp.int32, (_BLK, 1), 0)
    o_ref[...] = jnp.broadcast_to(wait_ref[...], (_BLK, _D))

    def leaky(x):
        return jnp.where(x >= 0, x, 0.01 * x)

    def dot(x, w):
        return jnp.dot(x, w, preferred_element_type=jnp.float32)

    @pl.when((b1 < end) & (b2 > start) & (b2 > b1))
    def _pick():
        h = leaky(dot(_unpack_pairs(a_ref[...]), pwa[...])
                  + dot(_unpack_pairs(f_ref[...]), pwf[...])
                  + dot(_unpack_pairs(t_ref[...]), pwt[...])
                  + dot(_unpack_pairs(m_ref[...]), pwm[...])
                  + pb1[...])
        out = dot(h, pw2[...]) + pb2[...]
        mask = (rows >= b1) & (rows < b2)
        o_ref[...] = jnp.where(mask, out, o_ref[...])

    @pl.when((b2 < end) & (b3 > start) & (b3 > b2))
    def _tr():
        h = leaky(dot(_unpack_pairs(a_ref[...]), twa[...])
                  + dot(_unpack_pairs(m_ref[...]), twm[...])
                  + tb1[...])
        out = dot(h, tw2[...]) + tb2[...]
        mask = (rows >= b2) & (rows < b3)
        o_ref[...] = jnp.where(mask, out, o_ref[...])

    @pl.when(b3 < end)
    def _mv():
        h = leaky(dot(_unpack_pairs(a_ref[...]), mwa[...])
                  + dot(_unpack_pairs(m_ref[...]), mwm[...])
                  + mb1[...])
        out = dot(h, mw2[...]) + mb2[...]
        mask = rows >= b3
        o_ref[...] = jnp.where(mask, out, o_ref[...])


def _tc_stage(bnd, A, M, F, Tt, pW1, pb1, pW2, pb2,
              tW1, tb1, tW2, tb2, mW1, mb1, mW2, mb2, wait_emb):
    nblk = _T // _BLK

    def am_map(i, b):
        lo = b[0] // _BLK
        return (jnp.minimum(jnp.maximum(i, lo), nblk - 1), 0)

    def ft_map(i, b):
        lo = b[0] // _BLK
        hi = jnp.maximum(lo, jnp.maximum(b[0], b[1] - 1) // _BLK)
        return (jnp.minimum(jnp.clip(i, lo, hi), nblk - 1), 0)

    w1_blk = lambda j: pl.BlockSpec((_D, _H), lambda i, b, j=j: (j, 0))
    full = lambda r, c: pl.BlockSpec((r, c), lambda i, b: (0, 0))
    in_specs = [
        pl.BlockSpec((_BLK, _D // 2), am_map),
        pl.BlockSpec((_BLK, _D // 2), am_map),
        pl.BlockSpec((_BLK, _D // 2), ft_map),
        pl.BlockSpec((_BLK, _D // 2), ft_map),
        w1_blk(0), w1_blk(1), w1_blk(2), w1_blk(3),
        full(1, _H), full(_H, _D), full(1, _D),
        w1_blk(0), w1_blk(1),
        full(1, _H), full(_H, _D), full(1, _D),
        w1_blk(0), w1_blk(1),
        full(1, _H), full(_H, _D), full(1, _D),
        full(1, _D),
    ]
    grid_spec = pltpu.PrefetchScalarGridSpec(
        num_scalar_prefetch=1,
        grid=(nblk,),
        in_specs=in_specs,
        out_specs=pl.BlockSpec((_BLK, _D), lambda i, b: (i, 0)),
    )
    return pl.pallas_call(
        _mlp_body,
        grid_spec=grid_spec,
        out_shape=jax.ShapeDtypeStruct((_T, _D), jnp.float32),
    )(bnd, A, M, F, Tt,
      pW1, pW1, pW1, pW1, pb1.reshape(1, _H), pW2, pb2.reshape(1, _D),
      tW1, tW1, tb1.reshape(1, _H), tW2, tb2.reshape(1, _D),
      mW1, mW1, mb1.reshape(1, _H), mW2, mb2.reshape(1, _D),
      wait_emb.reshape(1, _D))


def _plan(action_type):
    at = action_type.astype(jnp.int32)
    is1, is2, is3 = (at == 1), (at == 2), (at == 3)
    cs = jnp.cumsum(jnp.stack(
        [is1.astype(jnp.int32) + (is2.astype(jnp.int32) << 14),
         is3.astype(jnp.int32)]), axis=1)
    cs12, cs3 = cs[0], cs[1]
    c1 = cs12 & 0x3FFF
    c2 = cs12 >> 14
    c3 = cs3
    t1 = jnp.arange(1, _T + 1, dtype=jnp.int32)
    c0 = t1 - c1 - c2 - c3
    n0, n1, n2 = c0[-1], c1[-1], c2[-1]
    b1 = n0
    b2 = n0 + n1
    b3 = n0 + n1 + n2
    bnd = jnp.stack([b1, b2, b3]).astype(jnp.int32)
    rank = (jnp.where(is1, c1, 0) + jnp.where(is2, c2, 0)
            + jnp.where(is3, c3, 0)
            + jnp.where(at == 0, c0, 0)) - 1
    group_start = (jnp.where(is1, b1, 0) + jnp.where(is2, b2, 0)
                   + jnp.where(is3, b3, 0))
    pos = (group_start + rank).astype(jnp.int32)
    return pos, bnd


def kernel(op_emb, machine_emb, agv_emb, action_type, agv_idx, op_from_idx,
           op_to_idx, machine_idx, wait_emb, pick_W1, pick_b1, pick_W2,
           pick_b2, tr_W1, tr_b1, tr_W2, tr_b2, mv_W1, mv_b1, mv_W2, mv_b2):
    pos, bnd = _plan(action_type)
    idxmat = jnp.stack([agv_idx.astype(jnp.int32),
                        machine_idx.astype(jnp.int32),
                        op_from_idx.astype(jnp.int32),
                        op_to_idx.astype(jnp.int32)])
    A, M, F, Tt = _sc_gather_scatter4(
        _pack_bf16_pairs(op_emb), _pack_bf16_pairs(machine_emb),
        _pack_bf16_pairs(agv_emb), idxmat, pos)
    out_sorted = _tc_stage(bnd, A, M, F, Tt, pick_W1, pick_b1, pick_W2,
                           pick_b2, tr_W1, tr_b1, tr_W2, tr_b2,
                           mv_W1, mv_b1, mv_W2, mv_b2, wait_emb)
    return _sc_gather_rows(out_sorted, pos)


# speedup vs baseline: 1.0315x; 1.0315x over previous
import functools

import jax
import jax.numpy as jnp
from jax import lax
from jax.experimental import pallas as pl
from jax.experimental.pallas import tpu as pltpu
from jax.experimental.pallas import tpu_sc as plsc

_T = 8192
_D = 256
_H = 512
_BLK = 2048

_NC, _NS = 2, 16
_NW = _NC * _NS
_PERW = _T // _NW
_CH = 128
_NCH = _PERW // _CH


def _pack_bf16_pairs(x):
    xb = x.astype(jnp.bfloat16)
    h = x.shape[1] // 2
    lo = jax.lax.bitcast_convert_type(xb[:, :h], jnp.uint16).astype(jnp.int32)
    hi = jax.lax.bitcast_convert_type(xb[:, h:], jnp.uint16).astype(jnp.int32)
    return lo | (hi << 16)


def _unpack_pairs(x):
    lo = jax.lax.bitcast_convert_type(x << 16, jnp.float32)
    hi = jax.lax.bitcast_convert_type(x & jnp.int32(-65536), jnp.float32)
    return jnp.concatenate([lo, hi], axis=1)


_NB = 4


def _sc_gather_scatter4(op_emb, machine_emb, agv_emb, idxmat, pos):
    mesh = plsc.VectorSubcoreMesh(core_axis_name="c", subcore_axis_name="s")

    @functools.partial(
        pl.kernel, mesh=mesh,
        out_type=(jax.ShapeDtypeStruct((_T, _D // 2), jnp.int32),) * 4,
        scratch_types=[
            pltpu.VMEM((4, _PERW), jnp.int32),
            pltpu.VMEM((_CH,), jnp.int32),
            pltpu.VMEM((_CH,), jnp.int32),
        ] + [pltpu.VMEM((_CH, _D // 2), jnp.int32)] * _NB
          + [pltpu.SemaphoreType.DMA] * (2 * _NB),
    )
    def k(op_t, mach_t, agv_t, idxm_h, pos_h,
          oa, om, of_, ot,
          idxm_v, pos0, pos1, *bufs_and_sems):
        rowsb = bufs_and_sems[:_NB]
        sg = bufs_and_sems[_NB:2 * _NB]
        ss = bufs_and_sems[2 * _NB:3 * _NB]
        wid = lax.axis_index("s") * _NC + lax.axis_index("c")
        base = wid * _PERW
        posb = (pos0, pos1)
        pltpu.sync_copy(idxm_h.at[:, pl.ds(base, _PERW)], idxm_v)
        for c in range(_NCH):
            pltpu.sync_copy(pos_h.at[pl.ds(base + c * _CH, _CH)], posb[c])
        tabs = (agv_t, mach_t, op_t, op_t)
        outs = (oa, om, of_, ot)
        steps = [(c, f) for c in range(_NCH) for f in range(4)]
        n = len(steps)
        pend_g = [None] * _NB
        pend_s = [None] * _NB

        def start_gather(s):
            c, f = steps[s]
            b = s % _NB
            pend_g[b] = pltpu.async_copy(
                tabs[f].at[idxm_v.at[f, pl.ds(c * _CH, _CH)]], rowsb[b], sg[b])

        for s in range(min(3, n)):
            start_gather(s)
        for s in range(n):
            c, f = steps[s]
            b = s % _NB
            pend_g[b].wait()
            pend_s[b] = pltpu.async_copy(rowsb[b], outs[f].at[posb[c]], ss[b])
            ns = s + 3
            if ns < n:
                nb = ns % _NB
                if pend_s[nb] is not None:
                    pend_s[nb].wait()
                    pend_s[nb] = None
                start_gather(ns)
        for b in range(_NB):
            if pend_s[b] is not None:
                pend_s[b].wait()

    return k(op_emb, machine_emb, agv_emb, idxmat, pos)


def _sc_gather_rows(table, idx):
    mesh = plsc.VectorSubcoreMesh(core_axis_name="c", subcore_axis_name="s")

    @functools.partial(
        pl.kernel, mesh=mesh,
        out_type=jax.ShapeDtypeStruct((_T, _D), jnp.float32),
        scratch_types=[
            pltpu.VMEM((_PERW,), jnp.int32),
            pltpu.VMEM((_CH, _D), jnp.float32),
            pltpu.VMEM((_CH, _D), jnp.float32),
            pltpu.SemaphoreType.DMA,
            pltpu.SemaphoreType.DMA,
            pltpu.SemaphoreType.DMA,
            pltpu.SemaphoreType.DMA,
        ],
    )
    def k(tab, ih, oh, idx_v, r0, r1, sg0, sg1, sw0, sw1):
        wid = lax.axis_index("s") * _NC + lax.axis_index("c")
        base = wid * _PERW
        pltpu.sync_copy(ih.at[pl.ds(base, _PERW)], idx_v)
        g0 = pltpu.async_copy(tab.at[idx_v.at[pl.ds(0, _CH)]], r0, sg0)
        g1 = pltpu.async_copy(tab.at[idx_v.at[pl.ds(_CH, _CH)]], r1, sg1)
        g0.wait()
        w0 = pltpu.async_copy(r0, oh.at[pl.ds(base, _CH), :], sw0)
        g1.wait()
        w1 = pltpu.async_copy(r1, oh.at[pl.ds(base + _CH, _CH), :], sw1)
        w0.wait()
        w1.wait()

    return k(table, idx)


def _mlp_body(bnd_ref, a_ref, m_ref, f_ref, t_ref,
              pwa, pwf, pwt, pwm, pb1, pw2, pb2,
              twa, twm, tb1, tw2, tb2,
              mwa, mwm, mb1, mw2, mb2, wait_ref, o_ref):
    start = pl.program_id(0) * _BLK
    end = start + _BLK
    b1, b2, b3 = bnd_ref[0], bnd_ref[1], bnd_ref[2]
    rows = start + lax.broadcasted_iota(jnp.int32, (_BLK, 1), 0)
    o_ref[...] = jnp.broadcast_to(wait_ref[...], (_BLK, _D))

    def leaky(x):
        return jnp.where(x >= 0, x, 0.01 * x)

    def dot(x, w):
        return jnp.dot(x, w, preferred_element_type=jnp.float32)

    @pl.when((b1 < end) & (b2 > start) & (b2 > b1))
    def _pick():
        h = leaky(dot(_unpack_pairs(a_ref[...]), pwa[...])
                  + dot(_unpack_pairs(f_ref[...]), pwf[...])
                  + dot(_unpack_pairs(t_ref[...]), pwt[...])
                  + dot(_unpack_pairs(m_ref[...]), pwm[...])
                  + pb1[...])
        out = dot(h, pw2[...]) + pb2[...]
        mask = (rows >= b1) & (rows < b2)
        o_ref[...] = jnp.where(mask, out, o_ref[...])

    @pl.when((b2 < end) & (b3 > start) & (b3 > b2))
    def _tr():
        h = leaky(dot(_unpack_pairs(a_ref[...]), twa[...])
                  + dot(_unpack_pairs(m_ref[...]), twm[...])
                  + tb1[...])
        out = dot(h, tw2[...]) + tb2[...]
        mask = (rows >= b2) & (rows < b3)
        o_ref[...] = jnp.where(mask, out, o_ref[...])

    @pl.when(b3 < end)
    def _mv():
        h = leaky(dot(_unpack_pairs(a_ref[...]), mwa[...])
                  + dot(_unpack_pairs(m_ref[...]), mwm[...])
                  + mb1[...])
        out = dot(h, mw2[...]) + mb2[...]
        mask = rows >= b3
        o_ref[...] = jnp.where(mask, out, o_ref[...])


def _tc_stage(bnd, A, M, F, Tt, pW1, pb1, pW2, pb2,
              tW1, tb1, tW2, tb2, mW1, mb1, mW2, mb2, wait_emb):
    nblk = _T // _BLK

    def am_map(i, b):
        lo = b[0] // _BLK
        return (jnp.minimum(jnp.maximum(i, lo), nblk - 1), 0)

    def ft_map(i, b):
        lo = b[0] // _BLK
        hi = jnp.maximum(lo, jnp.maximum(b[0], b[1] - 1) // _BLK)
        return (jnp.minimum(jnp.clip(i, lo, hi), nblk - 1), 0)

    w1_blk = lambda j: pl.BlockSpec((_D, _H), lambda i, b, j=j: (j, 0))
    full = lambda r, c: pl.BlockSpec((r, c), lambda i, b: (0, 0))
    in_specs = [
        pl.BlockSpec((_BLK, _D // 2), am_map),
        pl.BlockSpec((_BLK, _D // 2), am_map),
        pl.BlockSpec((_BLK, _D // 2), ft_map),
        pl.BlockSpec((_BLK, _D // 2), ft_map),
        w1_blk(0), w1_blk(1), w1_blk(2), w1_blk(3),
        full(1, _H), full(_H, _D), full(1, _D),
        w1_blk(0), w1_blk(1),
        full(1, _H), full(_H, _D), full(1, _D),
        w1_blk(0), w1_blk(1),
        full(1, _H), full(_H, _D), full(1, _D),
        full(1, _D),
    ]
    grid_spec = pltpu.PrefetchScalarGridSpec(
        num_scalar_prefetch=1,
        grid=(nblk,),
        in_specs=in_specs,
        out_specs=pl.BlockSpec((_BLK, _D), lambda i, b: (i, 0)),
    )
    return pl.pallas_call(
        _mlp_body,
        grid_spec=grid_spec,
        out_shape=jax.ShapeDtypeStruct((_T, _D), jnp.float32),
    )(bnd, A, M, F, Tt,
      pW1, pW1, pW1, pW1, pb1.reshape(1, _H), pW2, pb2.reshape(1, _D),
      tW1, tW1, tb1.reshape(1, _H), tW2, tb2.reshape(1, _D),
      mW1, mW1, mb1.reshape(1, _H), mW2, mb2.reshape(1, _D),
      wait_emb.reshape(1, _D))


def _plan(action_type):
    at = action_type.astype(jnp.int32)
    is1, is2, is3 = (at == 1), (at == 2), (at == 3)
    cs = jnp.cumsum(jnp.stack(
        [is1.astype(jnp.int32) + (is2.astype(jnp.int32) << 14),
         is3.astype(jnp.int32)]), axis=1)
    cs12, cs3 = cs[0], cs[1]
    c1 = cs12 & 0x3FFF
    c2 = cs12 >> 14
    c3 = cs3
    t1 = jnp.arange(1, _T + 1, dtype=jnp.int32)
    c0 = t1 - c1 - c2 - c3
    n0, n1, n2 = c0[-1], c1[-1], c2[-1]
    b1 = n0
    b2 = n0 + n1
    b3 = n0 + n1 + n2
    bnd = jnp.stack([b1, b2, b3]).astype(jnp.int32)
    rank = (jnp.where(is1, c1, 0) + jnp.where(is2, c2, 0)
            + jnp.where(is3, c3, 0)
            + jnp.where(at == 0, c0, 0)) - 1
    group_start = (jnp.where(is1, b1, 0) + jnp.where(is2, b2, 0)
                   + jnp.where(is3, b3, 0))
    pos = (group_start + rank).astype(jnp.int32)
    return pos, bnd


def kernel(op_emb, machine_emb, agv_emb, action_type, agv_idx, op_from_idx,
           op_to_idx, machine_idx, wait_emb, pick_W1, pick_b1, pick_W2,
           pick_b2, tr_W1, tr_b1, tr_W2, tr_b2, mv_W1, mv_b1, mv_W2, mv_b2):
    pos, bnd = _plan(action_type)
    idxmat = jnp.stack([agv_idx.astype(jnp.int32),
                        machine_idx.astype(jnp.int32),
                        op_from_idx.astype(jnp.int32),
                        op_to_idx.astype(jnp.int32)])
    A, M, F, Tt = _sc_gather_scatter4(
        _pack_bf16_pairs(op_emb), _pack_bf16_pairs(machine_emb),
        _pack_bf16_pairs(agv_emb), idxmat, pos)
    out_sorted = _tc_stage(bnd, A, M, F, Tt, pick_W1, pick_b1, pick_W2,
                           pick_b2, tr_W1, tr_b1, tr_W2, tr_b2,
                           mv_W1, mv_b1, mv_W2, mv_b2, wait_emb)
    return _sc_gather_rows(out_sorted, pos)


# speedup vs baseline: 1.0366x; 1.0050x over previous
import functools

import jax
import jax.numpy as jnp
from jax import lax
from jax.experimental import pallas as pl
from jax.experimental.pallas import tpu as pltpu
from jax.experimental.pallas import tpu_sc as plsc

_T = 8192
_D = 256
_H = 512
_BLK = 1024

_NC, _NS = 2, 16
_NW = _NC * _NS
_PERW = _T // _NW
_CH = 128
_NCH = _PERW // _CH


def _pack_bf16_pairs(x):
    xb = x.astype(jnp.bfloat16)
    h = x.shape[1] // 2
    lo = jax.lax.bitcast_convert_type(xb[:, :h], jnp.uint16).astype(jnp.int32)
    hi = jax.lax.bitcast_convert_type(xb[:, h:], jnp.uint16).astype(jnp.int32)
    return lo | (hi << 16)


def _unpack_pairs(x):
    lo = jax.lax.bitcast_convert_type(x << 16, jnp.float32)
    hi = jax.lax.bitcast_convert_type(x & jnp.int32(-65536), jnp.float32)
    return jnp.concatenate([lo, hi], axis=1)


_NB = 4


def _sc_gather_scatter4(op_emb, machine_emb, agv_emb, idxmat, pos):
    mesh = plsc.VectorSubcoreMesh(core_axis_name="c", subcore_axis_name="s")

    @functools.partial(
        pl.kernel, mesh=mesh,
        out_type=(jax.ShapeDtypeStruct((_T, _D // 2), jnp.int32),) * 4,
        scratch_types=[
            pltpu.VMEM((4, _PERW), jnp.int32),
            pltpu.VMEM((_CH,), jnp.int32),
            pltpu.VMEM((_CH,), jnp.int32),
        ] + [pltpu.VMEM((_CH, _D // 2), jnp.int32)] * _NB
          + [pltpu.SemaphoreType.DMA] * (2 * _NB),
    )
    def k(op_t, mach_t, agv_t, idxm_h, pos_h,
          oa, om, of_, ot,
          idxm_v, pos0, pos1, *bufs_and_sems):
        rowsb = bufs_and_sems[:_NB]
        sg = bufs_and_sems[_NB:2 * _NB]
        ss = bufs_and_sems[2 * _NB:3 * _NB]
        wid = lax.axis_index("s") * _NC + lax.axis_index("c")
        base = wid * _PERW
        posb = (pos0, pos1)
        pltpu.sync_copy(idxm_h.at[:, pl.ds(base, _PERW)], idxm_v)
        for c in range(_NCH):
            pltpu.sync_copy(pos_h.at[pl.ds(base + c * _CH, _CH)], posb[c])
        tabs = (agv_t, mach_t, op_t, op_t)
        outs = (oa, om, of_, ot)
        steps = [(c, f) for c in range(_NCH) for f in range(4)]
        n = len(steps)
        pend_g = [None] * _NB
        pend_s = [None] * _NB

        def start_gather(s):
            c, f = steps[s]
            b = s % _NB
            pend_g[b] = pltpu.async_copy(
                tabs[f].at[idxm_v.at[f, pl.ds(c * _CH, _CH)]], rowsb[b], sg[b])

        for s in range(min(3, n)):
            start_gather(s)
        for s in range(n):
            c, f = steps[s]
            b = s % _NB
            pend_g[b].wait()
            pend_s[b] = pltpu.async_copy(rowsb[b], outs[f].at[posb[c]], ss[b])
            ns = s + 3
            if ns < n:
                nb = ns % _NB
                if pend_s[nb] is not None:
                    pend_s[nb].wait()
                    pend_s[nb] = None
                start_gather(ns)
        for b in range(_NB):
            if pend_s[b] is not None:
                pend_s[b].wait()

    return k(op_emb, machine_emb, agv_emb, idxmat, pos)


def _sc_gather_rows(table, idx):
    mesh = plsc.VectorSubcoreMesh(core_axis_name="c", subcore_axis_name="s")

    @functools.partial(
        pl.kernel, mesh=mesh,
        out_type=jax.ShapeDtypeStruct((_T, _D), jnp.float32),
        scratch_types=[
            pltpu.VMEM((_PERW,), jnp.int32),
            pltpu.VMEM((_CH, _D), jnp.float32),
            pltpu.VMEM((_CH, _D), jnp.float32),
            pltpu.SemaphoreType.DMA,
            pltpu.SemaphoreType.DMA,
            pltpu.SemaphoreType.DMA,
            pltpu.SemaphoreType.DMA,
        ],
    )
    def k(tab, ih, oh, idx_v, r0, r1, sg0, sg1, sw0, sw1):
        wid = lax.axis_index("s") * _NC + lax.axis_index("c")
        base = wid * _PERW
        pltpu.sync_copy(ih.at[pl.ds(base, _PERW)], idx_v)
        g0 = pltpu.async_copy(tab.at[idx_v.at[pl.ds(0, _CH)]], r0, sg0)
        g1 = pltpu.async_copy(tab.at[idx_v.at[pl.ds(_CH, _CH)]], r1, sg1)
        g0.wait()
        w0 = pltpu.async_copy(r0, oh.at[pl.ds(base, _CH), :], sw0)
        g1.wait()
        w1 = pltpu.async_copy(r1, oh.at[pl.ds(base + _CH, _CH), :], sw1)
        w0.wait()
        w1.wait()

    return k(table, idx)


def _mlp_body(bnd_ref, a_ref, m_ref, f_ref, t_ref,
              pwa, pwf, pwt, pwm, pb1, pw2, pb2,
              twa, twm, tb1, tw2, tb2,
              mwa, mwm, mb1, mw2, mb2, wait_ref, o_ref):
    start = pl.program_id(0) * _BLK
    end = start + _BLK
    b1, b2, b3 = bnd_ref[0], bnd_ref[1], bnd_ref[2]
    rows = start + lax.broadcasted_iota(jnp.int32, (_BLK, 1), 0)
    o_ref[...] = jnp.broadcast_to(wait_ref[...], (_BLK, _D))

    def leaky(x):
        return jnp.where(x >= 0, x, 0.01 * x)

    def dot(x, w):
        return jnp.dot(x, w, preferred_element_type=jnp.float32)

    @pl.when((b1 < end) & (b2 > start) & (b2 > b1))
    def _pick():
        h = leaky(dot(_unpack_pairs(a_ref[...]), pwa[...])
                  + dot(_unpack_pairs(f_ref[...]), pwf[...])
                  + dot(_unpack_pairs(t_ref[...]), pwt[...])
                  + dot(_unpack_pairs(m_ref[...]), pwm[...])
                  + pb1[...])
        out = dot(h, pw2[...]) + pb2[...]
        mask = (rows >= b1) & (rows < b2)
        o_ref[...] = jnp.where(mask, out, o_ref[...])

    @pl.when((b2 < end) & (b3 > start) & (b3 > b2))
    def _tr():
        h = leaky(dot(_unpack_pairs(a_ref[...]), twa[...])
                  + dot(_unpack_pairs(m_ref[...]), twm[...])
                  + tb1[...])
        out = dot(h, tw2[...]) + tb2[...]
        mask = (rows >= b2) & (rows < b3)
        o_ref[...] = jnp.where(mask, out, o_ref[...])

    @pl.when(b3 < end)
    def _mv():
        h = leaky(dot(_unpack_pairs(a_ref[...]), mwa[...])
                  + dot(_unpack_pairs(m_ref[...]), mwm[...])
                  + mb1[...])
        out = dot(h, mw2[...]) + mb2[...]
        mask = rows >= b3
        o_ref[...] = jnp.where(mask, out, o_ref[...])


def _tc_stage(bnd, A, M, F, Tt, pW1, pb1, pW2, pb2,
              tW1, tb1, tW2, tb2, mW1, mb1, mW2, mb2, wait_emb):
    nblk = _T // _BLK

    def am_map(i, b):
        lo = b[0] // _BLK
        return (jnp.minimum(jnp.maximum(i, lo), nblk - 1), 0)

    def ft_map(i, b):
        lo = b[0] // _BLK
        hi = jnp.maximum(lo, jnp.maximum(b[0], b[1] - 1) // _BLK)
        return (jnp.minimum(jnp.clip(i, lo, hi), nblk - 1), 0)

    w1_blk = lambda j: pl.BlockSpec((_D, _H), lambda i, b, j=j: (j, 0))
    full = lambda r, c: pl.BlockSpec((r, c), lambda i, b: (0, 0))
    in_specs = [
        pl.BlockSpec((_BLK, _D // 2), am_map),
        pl.BlockSpec((_BLK, _D // 2), am_map),
        pl.BlockSpec((_BLK, _D // 2), ft_map),
        pl.BlockSpec((_BLK, _D // 2), ft_map),
        w1_blk(0), w1_blk(1), w1_blk(2), w1_blk(3),
        full(1, _H), full(_H, _D), full(1, _D),
        w1_blk(0), w1_blk(1),
        full(1, _H), full(_H, _D), full(1, _D),
        w1_blk(0), w1_blk(1),
        full(1, _H), full(_H, _D), full(1, _D),
        full(1, _D),
    ]
    grid_spec = pltpu.PrefetchScalarGridSpec(
        num_scalar_prefetch=1,
        grid=(nblk,),
        in_specs=in_specs,
        out_specs=pl.BlockSpec((_BLK, _D), lambda i, b: (i, 0)),
    )
    return pl.pallas_call(
        _mlp_body,
        grid_spec=grid_spec,
        out_shape=jax.ShapeDtypeStruct((_T, _D), jnp.float32),
    )(bnd, A, M, F, Tt,
      pW1, pW1, pW1, pW1, pb1.reshape(1, _H), pW2, pb2.reshape(1, _D),
      tW1, tW1, tb1.reshape(1, _H), tW2, tb2.reshape(1, _D),
      mW1, mW1, mb1.reshape(1, _H), mW2, mb2.reshape(1, _D),
      wait_emb.reshape(1, _D))


def _plan(action_type):
    at = action_type.astype(jnp.int32)
    is1, is2, is3 = (at == 1), (at == 2), (at == 3)
    cs = jnp.cumsum(jnp.stack(
        [is1.astype(jnp.int32) + (is2.astype(jnp.int32) << 14),
         is3.astype(jnp.int32)]), axis=1)
    cs12, cs3 = cs[0], cs[1]
    c1 = cs12 & 0x3FFF
    c2 = cs12 >> 14
    c3 = cs3
    t1 = jnp.arange(1, _T + 1, dtype=jnp.int32)
    c0 = t1 - c1 - c2 - c3
    n0, n1, n2 = c0[-1], c1[-1], c2[-1]
    b1 = n0
    b2 = n0 + n1
    b3 = n0 + n1 + n2
    bnd = jnp.stack([b1, b2, b3]).astype(jnp.int32)
    rank = (jnp.where(is1, c1, 0) + jnp.where(is2, c2, 0)
            + jnp.where(is3, c3, 0)
            + jnp.where(at == 0, c0, 0)) - 1
    group_start = (jnp.where(is1, b1, 0) + jnp.where(is2, b2, 0)
                   + jnp.where(is3, b3, 0))
    pos = (group_start + rank).astype(jnp.int32)
    return pos, bnd


def kernel(op_emb, machine_emb, agv_emb, action_type, agv_idx, op_from_idx,
           op_to_idx, machine_idx, wait_emb, pick_W1, pick_b1, pick_W2,
           pick_b2, tr_W1, tr_b1, tr_W2, tr_b2, mv_W1, mv_b1, mv_W2, mv_b2):
    pos, bnd = _plan(action_type)
    idxmat = jnp.stack([agv_idx.astype(jnp.int32),
                        machine_idx.astype(jnp.int32),
                        op_from_idx.astype(jnp.int32),
                        op_to_idx.astype(jnp.int32)])
    A, M, F, Tt = _sc_gather_scatter4(
        _pack_bf16_pairs(op_emb), _pack_bf16_pairs(machine_emb),
        _pack_bf16_pairs(agv_emb), idxmat, pos)
    out_sorted = _tc_stage(bnd, A, M, F, Tt, pick_W1, pick_b1, pick_W2,
                           pick_b2, tr_W1, tr_b1, tr_W2, tr_b2,
                           mv_W1, mv_b1, mv_W2, mv_b2, wait_emb)
    return _sc_gather_rows(out_sorted, pos)
